# Initial kernel scaffold; baseline (speedup 1.0000x reference)
import jax
import jax.numpy as jnp
from jax.experimental import pallas as pl


def kernel(x, edge_index, edge_weight, Wc, bc, Wm, bm):
    raise NotImplementedError("write your pallas kernel here")


# speedup vs baseline: 4.7095x; 4.7095x over previous
import functools

import jax
import jax.numpy as jnp
from jax import lax
from jax.experimental import pallas as pl
from jax.experimental.pallas import tpu as pltpu
from jax.experimental.pallas import tpu_sc as plsc

NC = 2
NS = 16
LANES = 16
CHUNK = 128


@functools.lru_cache(maxsize=None)
def _make_sc_edge_layer(n_nodes: int, n_edges: int, d: int):
    nw = NC * NS
    assert n_edges % nw == 0
    e_per_w = n_edges // nw
    n_full = e_per_w // CHUNK
    tail = e_per_w - n_full * CHUNK
    assert tail % 8 == 0
    stripe = (n_nodes // (8 * NS)) * 8
    last_stripe = n_nodes - stripe * (NS - 1)
    d_vecs = d // LANES

    mesh = plsc.VectorSubcoreMesh(
        core_axis_name="c", subcore_axis_name="s", num_cores=NC, num_subcores=NS
    )

    @functools.partial(
        pl.kernel,
        out_type=jax.ShapeDtypeStruct((NC, n_nodes, d), jnp.float32),
        mesh=mesh,
        scratch_types=[
            pltpu.VMEM_SHARED((n_nodes, d), jnp.float32),
            pltpu.VMEM((CHUNK,), jnp.int32),
            pltpu.VMEM((CHUNK,), jnp.int32),
            pltpu.VMEM((CHUNK,), jnp.float32),
            pltpu.VMEM((CHUNK, d), jnp.float32),
            pltpu.VMEM((16,), jnp.int32),
            pltpu.VMEM((16,), jnp.int32),
            pltpu.VMEM((16,), jnp.float32),
            pltpu.VMEM((16, d), jnp.float32),
            pltpu.SemaphoreType.DMA,
        ],
    )
    def sc_layer(h_hbm, src_hbm, dst_hbm, ew_hbm, zeros_hbm, out_hbm,
                 acc, src_v, dst_v, ew_v, rows_v, src_t, dst_t, ew_t, rows_t,
                 sem):
        cid = lax.axis_index("c")
        sid = lax.axis_index("s")
        wid = cid * NS + sid
        row_base = sid * stripe

        @pl.when(sid < NS - 1)
        def _zero_acc():
            pltpu.sync_copy(zeros_hbm.at[pl.ds(0, stripe)],
                            acc.at[pl.ds(row_base, stripe)])

        @pl.when(sid == NS - 1)
        def _zero_acc_last():
            pltpu.sync_copy(zeros_hbm.at[pl.ds(0, last_stripe)],
                            acc.at[pl.ds(row_base, last_stripe)])

        plsc.subcore_barrier()

        edge_base = wid * e_per_w

        def do_chunk(off, srcr, dstr, ewr, rowsr, count):
            pltpu.sync_copy(src_hbm.at[pl.ds(off, count)], srcr)
            pltpu.sync_copy(dst_hbm.at[pl.ds(off, count)], dstr)
            pltpu.sync_copy(ew_hbm.at[pl.ds(off, count)], ewr)
            pltpu.async_copy(h_hbm.at[srcr], rowsr, sem).wait()

            @pl.loop(0, count // LANES)
            def _scale(g):
                w16 = ewr[pl.ds(g * LANES, LANES)]
                for k in range(LANES):
                    w = w16[k]
                    e = g * LANES + k
                    for j in range(d_vecs):
                        sl = pl.ds(j * LANES, LANES)
                        rowsr[e, sl] = rowsr[e, sl] * w

            pltpu.sync_copy(rowsr, acc.at[dstr], add=True)

        @pl.loop(0, n_full)
        def _edge_loop(i):
            do_chunk(edge_base + i * CHUNK, src_v, dst_v, ew_v, rows_v, CHUNK)

        if tail:
            assert tail == 16
            do_chunk(edge_base + n_full * CHUNK, src_t, dst_t, ew_t, rows_t, tail)

        plsc.subcore_barrier()

        @pl.when(sid < NS - 1)
        def _writeout():
            pltpu.sync_copy(acc.at[pl.ds(row_base, stripe)],
                            out_hbm.at[cid].at[pl.ds(row_base, stripe)])

        @pl.when(sid == NS - 1)
        def _writeout_last():
            pltpu.sync_copy(acc.at[pl.ds(row_base, last_stripe)],
                            out_hbm.at[cid].at[pl.ds(row_base, last_stripe)])

    return sc_layer


@functools.lru_cache(maxsize=None)
def _make_tc_dense_layer(n_nodes: int, d: int):
    blk = 1000
    assert n_nodes % blk == 0
    grid = n_nodes // blk

    def body(p_ref, wc_ref, bc_ref, wm_ref, bm_ref, o_ref):
        agg = p_ref[0] + p_ref[1]
        t = jnp.dot(agg, wc_ref[...], preferred_element_type=jnp.float32)
        t = t + bc_ref[...]
        y = jnp.dot(t, wm_ref[...], preferred_element_type=jnp.float32)
        y = y + bm_ref[...]
        o_ref[...] = jnp.where(y > 0, y, 0.01 * y)

    return pl.pallas_call(
        body,
        grid=(grid,),
        in_specs=[
            pl.BlockSpec((NC, blk, d), lambda i: (0, i, 0)),
            pl.BlockSpec((d, d), lambda i: (0, 0)),
            pl.BlockSpec((1, d), lambda i: (0, 0)),
            pl.BlockSpec((d, d), lambda i: (0, 0)),
            pl.BlockSpec((1, d), lambda i: (0, 0)),
        ],
        out_specs=pl.BlockSpec((blk, d), lambda i: (i, 0)),
        out_shape=jax.ShapeDtypeStruct((n_nodes, d), jnp.float32),
    )


def kernel(x, edge_index, edge_weight, Wc, bc, Wm, bm):
    n, d = x.shape
    e = edge_weight.shape[0]
    src = edge_index[0].astype(jnp.int32)
    dst = edge_index[1].astype(jnp.int32)
    ew = edge_weight.astype(jnp.float32)

    sc_layer = _make_sc_edge_layer(n, e, d)
    tc_layer = _make_tc_dense_layer(n, d)
    nz = n - (n // (8 * NS)) * 8 * (NS - 1)
    zeros = jnp.zeros((nz, d), jnp.float32)

    h = x
    for l in range(Wc.shape[0]):
        parts = sc_layer(h, src, dst, ew, zeros)
        h = tc_layer(parts, Wc[l], bc[l].reshape(1, d), Wm[l], bm[l].reshape(1, d))
    return h
